# baseline (speedup 1.0000x reference)
import functools

import numpy as np
import jax
import jax.numpy as jnp
from jax import lax
from jax.experimental import pallas as pl
from jax.experimental.pallas import tpu as pltpu
from jax.experimental.pallas import tpu_sc as plsc

_T, _B, _C = 4096, 16, 192
_VIS_T = _T - int(_T * 0.75)
_NC, _NS = 2, 16
_NW = _NC * _NS
_CT = _C // 8
_CTW = _CT // 2
_LANES = 16
_NBANKS = 16


def _color_classes(tv):
    edges = [(int(t), p) for p, t in enumerate(tv)]
    assign, su, du = [], [], []
    for t, p in edges:
        sb, db = t % _NBANKS, p % _NBANKS
        for ci in range(len(assign)):
            if (len(assign[ci]) < _LANES and sb not in su[ci]
                    and db not in du[ci]):
                break
        else:
            ci = len(assign)
            assign.append([])
            su.append(set())
            du.append(set())
        assign[ci].append((t, p))
        su[ci].add(sb)
        du[ci].add(db)
    n = len(edges)
    for ci in range(len(assign)):
        k = (ci * 131) % n
        while len(assign[ci]) < _LANES:
            for j in range(n):
                t, p = edges[(k + j) % n]
                if t % _NBANKS not in su[ci] and p % _NBANKS not in du[ci]:
                    break
            else:
                for j in range(n):
                    t, p = edges[(k + j) % n]
                    if p % _NBANKS not in du[ci]:
                        break
                else:
                    raise AssertionError("unfillable class")
            assign[ci].append((t, p))
            su[ci].add(t % _NBANKS)
            du[ci].add(p % _NBANKS)
            k = (k + j + 1) % n
    return assign


@functools.cache
def _host_indexes():
    with jax.default_device(jax.local_devices(backend="cpu")[0]):
        base = jax.random.key(42)
        perms = [jax.random.permutation(jax.random.fold_in(base, b), _T)
                 for b in range(_B)]
        fwd = np.asarray(jnp.stack(perms, axis=-1).astype(jnp.int32))
    bwd = np.argsort(fwd, axis=0).astype(np.int32)
    per_b = [_color_classes(fwd[:_VIS_T, b]) for b in range(_B)]
    ngroups = -(-max(len(cl) for cl in per_b) // 4) * 4
    nrows = ngroups // 4
    nrows_pad = -(-nrows // 8) * 8
    cidx = np.zeros((_NW, nrows_pad, 128), np.int32)
    for w in range(_NW):
        classes = per_b[w // 2]
        for g in range(ngroups):
            cl = classes[g] if g < len(classes) else classes[0]
            row, off = g // 4, (g % 4) * 32
            cidx[w, row, off:off + 16] = [t for t, _ in cl]
            cidx[w, row, off + 16:off + 32] = [p for _, p in cl]
    return fwd, bwd, cidx, ngroups, nrows_pad


_FWD_NP, _BWD_NP, _CIDX_NP, _NG, _NR = _host_indexes()


@functools.cache
def _build_gather():
    @functools.partial(
        pl.kernel,
        mesh=plsc.VectorSubcoreMesh(core_axis_name="c", subcore_axis_name="s"),
        compiler_params=pltpu.CompilerParams(use_tc_tiling_on_sc=True,
                                             needs_layout_passes=False,
                                             disable_bounds_checks=True,
                                             disable_semaphore_checks=True,
                                             skip_device_barrier=True),
        out_type=(
            jax.ShapeDtypeStruct((_B, _C, _VIS_T), jnp.float32),
            jax.ShapeDtypeStruct((_B, _T), jnp.int32),
            jax.ShapeDtypeStruct((_B, _T), jnp.int32),
        ),
        scratch_types=[
            pltpu.VMEM((_NR, 128), jnp.int32),
            pltpu.VMEM((8, _T), jnp.float32),
            pltpu.VMEM((8, _T), jnp.float32),
            pltpu.VMEM((8, _VIS_T), jnp.float32),
            pltpu.VMEM((8, _VIS_T), jnp.float32),
            pltpu.VMEM((2, 8, 256), jnp.int32),
            pltpu.SemaphoreType.DMA,
            pltpu.SemaphoreType.DMA,
            pltpu.SemaphoreType.DMA,
            pltpu.SemaphoreType.DMA,
            pltpu.SemaphoreType.DMA,
            pltpu.SemaphoreType.DMA,
            pltpu.SemaphoreType.DMA,
        ],
    )
    def _gather(tbl_hbm, fwd_hbm, bwd_hbm, cidx_hbm,
                vis_hbm, fwd_out, bwd_out,
                idx_v, in_a, in_b, out_a, out_b, pf_v,
                sem_a, sem_b, sem_oa, sem_ob, sem_ix, sem_pf, sem_pb):
        wid = lax.axis_index("s") * _NC + lax.axis_index("c")
        b = wid // 2
        base = (wid % 2) * _CTW

        lix = pltpu.async_copy(cidx_hbm.at[wid], idx_v, sem_ix)

        def in_slab(ct):
            return tbl_hbm.at[b, pl.ds(ct * 8, 8), :]

        def out_slab(ct):
            return vis_hbm.at[b, pl.ds(ct * 8, 8), :]

        pltpu.async_copy(in_slab(base), in_a, sem_a)
        pltpu.async_copy(in_slab(base + 1), in_b, sem_b)

        r0 = (wid % 2) * 8
        c0 = (wid // 2) * 256
        lpf = pltpu.async_copy(fwd_hbm.at[pl.ds(r0, 8), pl.ds(c0, 256)],
                               pf_v.at[0], sem_pf)
        lpb = pltpu.async_copy(bwd_hbm.at[pl.ds(r0, 8), pl.ds(c0, 256)],
                               pf_v.at[1], sem_pb)
        lix.wait()

        def compute(in_v, out_v):
            for g in range(_NG):
                row, off = g // 4, (g % 4) * 32
                sv = idx_v[row, pl.ds(off, _LANES)]
                dv = idx_v[row, pl.ds(off + _LANES, _LANES)]
                for r in range(8):
                    rv = jnp.full((_LANES,), r, jnp.int32)
                    plsc.store_scatter(out_v, [rv, dv],
                                       plsc.load_gather(in_v, [rv, sv]))

        def step(m, ct, in_v, out_v, sem_i, sem_o):
            pltpu.make_async_copy(in_slab(ct), in_v, sem_i).wait()

            @pl.when(m > 0)
            def _():
                pltpu.make_async_copy(out_v, out_slab(ct - 2), sem_o).wait()

            compute(in_v, out_v)
            pltpu.async_copy(out_v, out_slab(ct), sem_o)

            @pl.when(m < _CTW // 2 - 1)
            def _():
                pltpu.async_copy(in_slab(ct + 2), in_v, sem_i)

        def body(m, carry):
            step(m, base + 2 * m, in_a, out_a, sem_a, sem_oa)
            step(m, base + 2 * m + 1, in_b, out_b, sem_b, sem_ob)
            return carry

        lax.fori_loop(0, _CTW // 2, body, 0)
        lpf.wait()
        pltpu.async_copy(pf_v.at[0], fwd_out.at[pl.ds(r0, 8), pl.ds(c0, 256)],
                         sem_pf)
        lpb.wait()
        pltpu.async_copy(pf_v.at[1], bwd_out.at[pl.ds(r0, 8), pl.ds(c0, 256)],
                         sem_pb)
        pltpu.make_async_copy(out_a, out_slab(base + _CTW - 2), sem_oa).wait()
        pltpu.make_async_copy(out_b, out_slab(base + _CTW - 1), sem_ob).wait()
        pltpu.make_async_copy(pf_v.at[0],
                              fwd_out.at[pl.ds(r0, 8), pl.ds(c0, 256)],
                              sem_pf).wait()
        pltpu.make_async_copy(pf_v.at[1],
                              bwd_out.at[pl.ds(r0, 8), pl.ds(c0, 256)],
                              sem_pb).wait()

    return _gather


def kernel(patches):
    tblT = jnp.transpose(patches, (1, 2, 0))
    visT, fwdT, bwdT = _build_gather()(
        tblT, jnp.asarray(_FWD_NP.T), jnp.asarray(_BWD_NP.T),
        jnp.asarray(_CIDX_NP))
    vis = jnp.transpose(visT, (2, 0, 1))
    return (vis, fwdT.T, bwdT.T, jnp.int32(_VIS_T))


# speedup vs baseline: 1.3693x; 1.3693x over previous
import functools

import numpy as np
import jax
import jax.numpy as jnp
from jax import lax
from jax.experimental import pallas as pl
from jax.experimental.pallas import tpu as pltpu
from jax.experimental.pallas import tpu_sc as plsc

_T, _B, _C = 4096, 16, 192
_VIS_T = _T - int(_T * 0.75)
_NC, _NS = 2, 16
_NW = _NC * _NS
_CT = _C // 8
_CTW = _CT // 2
_LANES = 16


@functools.cache
def _host_indexes():
    with jax.default_device(jax.local_devices(backend="cpu")[0]):
        base = jax.random.key(42)
        perms = [jax.random.permutation(jax.random.fold_in(base, b), _T)
                 for b in range(_B)]
        fwd = np.asarray(jnp.stack(perms, axis=-1).astype(jnp.int32))
    bwd = np.argsort(fwd, axis=0).astype(np.int32)
    gidx = np.stack([fwd[:_VIS_T, w // 2].reshape(8, 128)
                     for w in range(_NW)]).astype(np.int32)
    return fwd, bwd, gidx


_FWD_NP, _BWD_NP, _GIDX_NP = _host_indexes()


@functools.cache
def _build_gather():
    @functools.partial(
        pl.kernel,
        mesh=plsc.VectorSubcoreMesh(core_axis_name="c", subcore_axis_name="s"),
        compiler_params=pltpu.CompilerParams(use_tc_tiling_on_sc=True,
                                             needs_layout_passes=False,
                                             disable_bounds_checks=True,
                                             disable_semaphore_checks=True,
                                             skip_device_barrier=True),
        out_type=(
            jax.ShapeDtypeStruct((_B, _C, _VIS_T), jnp.float32),
            jax.ShapeDtypeStruct((_B, _T), jnp.int32),
            jax.ShapeDtypeStruct((_B, _T), jnp.int32),
        ),
        scratch_types=[
            pltpu.VMEM((8, 128), jnp.int32),
            pltpu.VMEM((8, _T), jnp.float32),
            pltpu.VMEM((8, _T), jnp.float32),
            pltpu.VMEM((8, _VIS_T), jnp.float32),
            pltpu.VMEM((8, _VIS_T), jnp.float32),
            pltpu.VMEM((2, 8, 256), jnp.int32),
            pltpu.SemaphoreType.DMA,
            pltpu.SemaphoreType.DMA,
            pltpu.SemaphoreType.DMA,
            pltpu.SemaphoreType.DMA,
            pltpu.SemaphoreType.DMA,
            pltpu.SemaphoreType.DMA,
            pltpu.SemaphoreType.DMA,
        ],
    )
    def _gather(tbl_hbm, fwd_hbm, bwd_hbm, gidx_hbm,
                vis_hbm, fwd_out, bwd_out,
                idx_v, in_a, in_b, out_a, out_b, pf_v,
                sem_a, sem_b, sem_oa, sem_ob, sem_ix, sem_pf, sem_pb):
        wid = lax.axis_index("s") * _NC + lax.axis_index("c")
        b = wid // 2
        base = (wid % 2) * _CTW

        lix = pltpu.async_copy(gidx_hbm.at[wid], idx_v, sem_ix)

        def in_slab(ct):
            return tbl_hbm.at[b, pl.ds(ct * 8, 8), :]

        def out_slab(ct):
            return vis_hbm.at[b, pl.ds(ct * 8, 8), :]

        pltpu.async_copy(in_slab(base), in_a, sem_a)
        pltpu.async_copy(in_slab(base + 1), in_b, sem_b)

        r0 = (wid % 2) * 8
        c0 = (wid // 2) * 256
        lpf = pltpu.async_copy(fwd_hbm.at[pl.ds(r0, 8), pl.ds(c0, 256)],
                               pf_v.at[0], sem_pf)
        lpb = pltpu.async_copy(bwd_hbm.at[pl.ds(r0, 8), pl.ds(c0, 256)],
                               pf_v.at[1], sem_pb)
        lix.wait()

        def compute(in_v, out_v):
            for k in range(_VIS_T // _LANES):
                tv = idx_v[k // 8, pl.ds((k % 8) * _LANES, _LANES)]
                for r in range(8):
                    rv = jnp.full((_LANES,), r, jnp.int32)
                    out_v[r, pl.ds(k * _LANES, _LANES)] = plsc.load_gather(
                        in_v, [rv, tv])

        def step(m, ct, in_v, out_v, sem_i, sem_o):
            pltpu.make_async_copy(in_slab(ct), in_v, sem_i).wait()

            @pl.when(m > 0)
            def _():
                pltpu.make_async_copy(out_v, out_slab(ct - 2), sem_o).wait()

            compute(in_v, out_v)
            pltpu.async_copy(out_v, out_slab(ct), sem_o)

            @pl.when(m < _CTW // 2 - 1)
            def _():
                pltpu.async_copy(in_slab(ct + 2), in_v, sem_i)

        def body(m, carry):
            step(m, base + 2 * m, in_a, out_a, sem_a, sem_oa)
            step(m, base + 2 * m + 1, in_b, out_b, sem_b, sem_ob)
            return carry

        lax.fori_loop(0, _CTW // 2, body, 0)
        lpf.wait()
        pltpu.async_copy(pf_v.at[0], fwd_out.at[pl.ds(r0, 8), pl.ds(c0, 256)],
                         sem_pf)
        lpb.wait()
        pltpu.async_copy(pf_v.at[1], bwd_out.at[pl.ds(r0, 8), pl.ds(c0, 256)],
                         sem_pb)
        pltpu.make_async_copy(out_a, out_slab(base + _CTW - 2), sem_oa).wait()
        pltpu.make_async_copy(out_b, out_slab(base + _CTW - 1), sem_ob).wait()
        pltpu.make_async_copy(pf_v.at[0],
                              fwd_out.at[pl.ds(r0, 8), pl.ds(c0, 256)],
                              sem_pf).wait()
        pltpu.make_async_copy(pf_v.at[1],
                              bwd_out.at[pl.ds(r0, 8), pl.ds(c0, 256)],
                              sem_pb).wait()

    return _gather


def kernel(patches):
    tblT = jnp.transpose(patches, (1, 2, 0))
    visT, fwdT, bwdT = _build_gather()(
        tblT, jnp.asarray(_FWD_NP.T), jnp.asarray(_BWD_NP.T),
        jnp.asarray(_GIDX_NP))
    vis = jnp.transpose(visT, (2, 0, 1))
    return (vis, fwdT.T, bwdT.T, jnp.int32(_VIS_T))
